# Initial kernel scaffold; baseline (speedup 1.0000x reference)
import jax
import jax.numpy as jnp
from jax.experimental import pallas as pl


def kernel(x, edge_index, edge_d, W1, b1, Wt, bt, Wp, bp):
    raise NotImplementedError("write your pallas kernel here")


# speedup vs baseline: 2.1157x; 2.1157x over previous
import functools

import jax
import jax.numpy as jnp
from jax import lax
from jax.experimental import pallas as pl
from jax.experimental.pallas import tpu as pltpu
from jax.experimental.pallas import tpu_sc as plsc

_N = 10000
_E = 320000
_D = 128

_NTILES = 32
_EPT = _E // _NTILES
_C1 = 80
_NC1 = _EPT // _C1
_RPS = 624

_EH = _E // 2
_C2 = 640
_NC2 = _EH // _C2
_NSUB = _C2 // 128
_NEG = -3.0e38

_mesh = plsc.VectorSubcoreMesh(core_axis_name="c", subcore_axis_name="s")
_sc_params = pltpu.CompilerParams(needs_layout_passes=False,
                                  use_tc_tiling_on_sc=False)


@functools.partial(
    pl.kernel,
    out_type=(jax.ShapeDtypeStruct((2, _N, _D), jnp.float32),
              jax.ShapeDtypeStruct((_NTILES, _N), jnp.float32)),
    mesh=_mesh,
    compiler_params=_sc_params,
    scratch_types=(
        pltpu.VMEM((_C1,), jnp.int32),
        pltpu.VMEM((_C1,), jnp.int32),
        pltpu.VMEM((_C1,), jnp.float32),
        pltpu.VMEM((_C1, _D), jnp.float32),
        pltpu.VMEM((_N,), jnp.float32),
        pltpu.VMEM_SHARED((_N, _D), jnp.float32),
        pltpu.SemaphoreType.DMA,
    ),
)
def _stage1(x_hbm, src_hbm, dst_hbm, d_hbm, acc_out, deg_out,
            src_v, dst_v, d_v, rows, deg_l, acc_sh, sem):
    c = lax.axis_index("c")
    s = lax.axis_index("s")
    wid = c * 16 + s
    zero16 = jnp.zeros((16,), jnp.float32)
    ones16 = jnp.ones((16,), jnp.float32)

    def zdeg(i, _):
        deg_l[pl.ds(i * 16, 16)] = zero16
        return 0
    lax.fori_loop(0, _N // 16, zdeg, 0)

    def zrows(j, _):
        for k in range(_D // 16):
            rows[j, pl.ds(k * 16, 16)] = zero16
        return 0
    lax.fori_loop(0, _C1, zrows, 0)

    r0 = s * _RPS
    for t in range(_RPS // _C1):
        pltpu.sync_copy(rows, acc_sh.at[pl.ds(r0 + t * _C1, _C1), :])
    rem = _RPS % _C1
    pltpu.sync_copy(rows.at[pl.ds(0, rem), :],
                    acc_sh.at[pl.ds(r0 + _RPS - rem, rem), :])

    @pl.when(s == 15)
    def _():
        pltpu.sync_copy(rows.at[pl.ds(0, _N - 16 * _RPS), :],
                        acc_sh.at[pl.ds(16 * _RPS, _N - 16 * _RPS), :])

    plsc.subcore_barrier()

    eb = wid * _EPT

    def chunk(i, _):
        base = eb + i * _C1
        pltpu.sync_copy(src_hbm.at[pl.ds(base, _C1)], src_v)
        pltpu.sync_copy(dst_hbm.at[pl.ds(base, _C1)], dst_v)
        pltpu.sync_copy(d_hbm.at[pl.ds(base, _C1)], d_v)
        pltpu.async_copy(x_hbm.at[src_v], rows, sem).wait()

        def scale(g, _):
            w16 = 1.0 - d_v[pl.ds(g * 16, 16)]
            for j in range(16):
                w = w16[j]
                e = g * 16 + j
                for k in range(_D // 16):
                    rows[e, pl.ds(k * 16, 16)] = rows[e, pl.ds(k * 16, 16)] * w
            return 0
        lax.fori_loop(0, _C1 // 16, scale, 0)

        for g in range(_C1 // 16):
            dv = dst_v[pl.ds(g * 16, 16)]
            plsc.addupdate_scatter(deg_l, [dv], ones16)

        pltpu.sync_copy(rows, acc_sh.at[dst_v], add=True)
        return 0
    lax.fori_loop(0, _NC1, chunk, 0)

    pltpu.sync_copy(deg_l, deg_out.at[wid])
    plsc.subcore_barrier()

    pltpu.sync_copy(acc_sh.at[pl.ds(r0, _RPS), :],
                    acc_out.at[c, pl.ds(r0, _RPS), :])

    @pl.when(s == 15)
    def _():
        pltpu.sync_copy(acc_sh.at[pl.ds(16 * _RPS, _N - 16 * _RPS), :],
                        acc_out.at[c, pl.ds(16 * _RPS, _N - 16 * _RPS), :])


_RB = 1000


def _tc_mats_body(acc_ref, deg_ref, x_ref, w1_ref, b1_ref, wt_ref, wp_ref,
                  bpt_ref, ht_ref, hpb_ref):
    a = acc_ref[0] + acc_ref[1]
    degs = jnp.sum(deg_ref[...], axis=0)
    h0 = a + degs * x_ref[...]
    h = jnp.maximum(
        lax.dot_general(h0, w1_ref[...], (((1,), (1,)), ((), ())),
                        preferred_element_type=jnp.float32) + b1_ref[...],
        0.0)
    ht_ref[...] = lax.dot_general(h, wt_ref[...], (((1,), (1,)), ((), ())),
                                  preferred_element_type=jnp.float32)
    hpb_ref[...] = lax.dot_general(h, wp_ref[...], (((1,), (1,)), ((), ())),
                                   preferred_element_type=jnp.float32) + bpt_ref[...]


def _tc_mats(acc, deg, x, W1, b1, Wt, Wp, bpt):
    return pl.pallas_call(
        _tc_mats_body,
        grid=(_N // _RB,),
        in_specs=[
            pl.BlockSpec((2, _RB, _D), lambda i: (0, i, 0)),
            pl.BlockSpec((_NTILES, _RB, 1), lambda i: (0, i, 0)),
            pl.BlockSpec((_RB, _D), lambda i: (i, 0)),
            pl.BlockSpec((_D, _D), lambda i: (0, 0)),
            pl.BlockSpec((1, _D), lambda i: (0, 0)),
            pl.BlockSpec((_D, _D), lambda i: (0, 0)),
            pl.BlockSpec((_D, _D), lambda i: (0, 0)),
            pl.BlockSpec((1, _D), lambda i: (0, 0)),
        ],
        out_specs=[
            pl.BlockSpec((_RB, _D), lambda i: (i, 0)),
            pl.BlockSpec((_RB, _D), lambda i: (i, 0)),
        ],
        out_shape=[
            jax.ShapeDtypeStruct((_N, _D), jnp.float32),
            jax.ShapeDtypeStruct((_N, _D), jnp.float32),
        ],
    )(acc, deg, x, W1, b1, Wt, Wp, bpt)


@functools.partial(
    pl.kernel,
    out_type=jax.ShapeDtypeStruct((2, 16, _N * 8), jnp.float32),
    mesh=_mesh,
    compiler_params=_sc_params,
    scratch_types=(
        pltpu.VMEM((_C2,), jnp.int32),
        pltpu.VMEM((_C2,), jnp.int32),
        pltpu.VMEM((_C2,), jnp.float32),
        pltpu.VMEM((_NSUB, 128), jnp.int32),
        pltpu.VMEM((_C2, 8), jnp.float32),
        pltpu.VMEM((_N * 8,), jnp.float32),
        pltpu.SemaphoreType.DMA,
    ),
)
def _stage2(htb_hbm, src_hbm, dst_hbm, d_hbm, mx_out,
            src_v, dst_v, d_v, sidx, rows, acc, sem):
    c = lax.axis_index("c")
    s = lax.axis_index("s")
    fb = s
    eh = c
    iota = lax.iota(jnp.int32, 16)
    hi = iota >= 8
    lanecol = jnp.where(hi, 15 - iota, iota)
    negv = jnp.full((16,), _NEG, jnp.float32)

    def ini(i, _):
        acc[pl.ds(i * 16, 16)] = negv
        return 0
    lax.fori_loop(0, _N * 8 // 16, ini, 0)

    fbN = fb * _N

    def chunk(i, _):
        base = eh * _EH + i * _C2
        pltpu.sync_copy(src_hbm.at[pl.ds(base, _C2)], src_v)
        pltpu.sync_copy(dst_hbm.at[pl.ds(base, _C2)], dst_v)
        pltpu.sync_copy(d_hbm.at[pl.ds(base, _C2)], d_v)

        def mkidx(t, _):
            sidx[t // 8, pl.ds((t % 8) * 16, 16)] = src_v[pl.ds(t * 16, 16)] + fbN
            return 0
        lax.fori_loop(0, _C2 // 16, mkidx, 0)

        cps = [pltpu.async_copy(htb_hbm.at[sidx.at[k]],
                                rows.at[pl.ds(k * 128, 128), :], sem)
               for k in range(_NSUB)]
        for cp in cps:
            cp.wait()

        hi_i32 = hi.astype(jnp.int32)

        def grp16(q, _):
            dst16 = dst_v[pl.ds(q * 16, 16)]
            d16 = d_v[pl.ds(q * 16, 16)]
            for p in range(8):
                dsta = dst16[2 * p]
                dstb = dst16[2 * p + 1]
                da = d16[2 * p]
                db = d16[2 * p + 1]
                rowsel = (q * 16 + 2 * p) + hi_i32
                rv = plsc.load_gather(rows, [rowsel, lanecol])
                val = jnp.where(hi, db, da) * rv
                dstv = jnp.where(hi, dstb, dsta)
                addr = dstv * 8 + lanecol
                sw = lax.rev(val, (0,))
                val = jnp.where(dsta == dstb, jnp.maximum(val, sw), val)
                cur = plsc.load_gather(acc, [addr])
                plsc.store_scatter(acc, [addr], jnp.maximum(cur, val))
            return 0
        lax.fori_loop(0, _C2 // 16, grp16, 0)
        return 0
    lax.fori_loop(0, _NC2, chunk, 0)

    pltpu.sync_copy(acc, mx_out.at[eh, fb])


def _tc_mean_body(m_ref, hpb_ref, out_ref):
    v = jnp.maximum(jnp.maximum(m_ref[0], m_ref[1]) + hpb_ref[...], 0.0)
    part = jnp.sum(v, axis=0, keepdims=True) * (1.0 / _N)

    @pl.when(pl.program_id(0) == 0)
    def _():
        out_ref[...] = part

    @pl.when(pl.program_id(0) != 0)
    def _():
        out_ref[...] = out_ref[...] + part


def _tc_mean(mt, hpb):
    return pl.pallas_call(
        _tc_mean_body,
        grid=(_N // _RB,),
        in_specs=[
            pl.BlockSpec((2, _RB, _D), lambda i: (0, i, 0)),
            pl.BlockSpec((_RB, _D), lambda i: (i, 0)),
        ],
        out_specs=pl.BlockSpec((1, _D), lambda i: (0, 0)),
        out_shape=jax.ShapeDtypeStruct((1, _D), jnp.float32),
    )(mt, hpb)


def kernel(x, edge_index, edge_d, W1, b1, Wt, bt, Wp, bp):
    src = edge_index[0]
    dst = edge_index[1]
    acc, deg = _stage1(x, src, dst, edge_d)
    ht, hpb = _tc_mats(acc, deg.reshape(_NTILES, _N, 1), x, W1,
                       b1.reshape(1, _D), Wt, Wp, (bt + bp).reshape(1, _D))
    htb = ht.reshape(_N, 16, 8).transpose(1, 0, 2).reshape(16 * _N, 8)
    m = _stage2(htb, src, dst, edge_d)
    mt = m.reshape(2, 16, _N, 8).transpose(0, 2, 1, 3).reshape(2, _N, _D)
    return _tc_mean(mt, hpb)


# speedup vs baseline: 2.4857x; 1.1749x over previous
import functools

import jax
import jax.numpy as jnp
from jax import lax
from jax.experimental import pallas as pl
from jax.experimental.pallas import tpu as pltpu
from jax.experimental.pallas import tpu_sc as plsc

_N = 10000
_E = 320000
_D = 128

_NTILES = 32
_EPT = _E // _NTILES
_C1 = 80
_NC1 = _EPT // _C1
_RPS = 624

_HALF = _N // 2
_PAD = 512
_C2 = 512
_NSUB = _C2 // 128
_TRASH = _HALF
_ACCR = 5008
_NEG = -3.0e38
_BIGDST = 1 << 29

_mesh = plsc.VectorSubcoreMesh(core_axis_name="c", subcore_axis_name="s")
_sc_params = pltpu.CompilerParams(needs_layout_passes=False,
                                  use_tc_tiling_on_sc=False)


@functools.partial(
    pl.kernel,
    out_type=(jax.ShapeDtypeStruct((2, _N, _D), jnp.float32),
              jax.ShapeDtypeStruct((_NTILES, _N), jnp.float32),
              jax.ShapeDtypeStruct((_E + _PAD,), jnp.int32),
              jax.ShapeDtypeStruct((_E + _PAD,), jnp.int32),
              jax.ShapeDtypeStruct((_E + _PAD,), jnp.float32),
              jax.ShapeDtypeStruct((_NTILES * 8,), jnp.int32)),
    mesh=_mesh,
    compiler_params=_sc_params,
    scratch_types=(
        pltpu.VMEM((_C1,), jnp.int32),
        pltpu.VMEM((_C1,), jnp.int32),
        pltpu.VMEM((_C1,), jnp.float32),
        pltpu.VMEM((_C1, _D), jnp.float32),
        pltpu.VMEM((_N,), jnp.float32),
        pltpu.VMEM((_EPT,), jnp.int32),
        pltpu.VMEM((_EPT,), jnp.int32),
        pltpu.VMEM((_EPT,), jnp.float32),
        pltpu.VMEM((16,), jnp.int32),
        pltpu.VMEM_SHARED((_N, _D), jnp.float32),
        pltpu.SemaphoreType.DMA,
    ),
)
def _stage1(x_hbm, src_hbm, dst_hbm, d_hbm,
            acc_out, deg_out, bsrc_out, bdst_out, bd_out, cnt_out,
            src_v, dst_v, d_v, rows, deg_l, stg_s, stg_t, stg_w, cvec,
            acc_sh, sem):
    c = lax.axis_index("c")
    s = lax.axis_index("s")
    wid = c * 16 + s
    zero16 = jnp.zeros((16,), jnp.float32)
    ones16 = jnp.ones((16,), jnp.float32)

    def zdeg(i, _):
        deg_l[pl.ds(i * 16, 16)] = zero16
        return 0
    lax.fori_loop(0, _N // 16, zdeg, 0)

    def zrows(j, _):
        for k in range(_D // 16):
            rows[j, pl.ds(k * 16, 16)] = zero16
        return 0
    lax.fori_loop(0, _C1, zrows, 0)

    r0 = s * _RPS
    for t in range(_RPS // _C1):
        pltpu.sync_copy(rows, acc_sh.at[pl.ds(r0 + t * _C1, _C1), :])
    rem = _RPS % _C1
    pltpu.sync_copy(rows.at[pl.ds(0, rem), :],
                    acc_sh.at[pl.ds(r0 + _RPS - rem, rem), :])

    @pl.when(s == 15)
    def _():
        pltpu.sync_copy(rows.at[pl.ds(0, _N - 16 * _RPS), :],
                        acc_sh.at[pl.ds(16 * _RPS, _N - 16 * _RPS), :])

    plsc.subcore_barrier()

    eb = wid * _EPT

    def chunk(i, cv):
        base = eb + i * _C1
        pltpu.sync_copy(src_hbm.at[pl.ds(base, _C1)], src_v)
        pltpu.sync_copy(dst_hbm.at[pl.ds(base, _C1)], dst_v)
        pltpu.sync_copy(d_hbm.at[pl.ds(base, _C1)], d_v)
        pltpu.async_copy(x_hbm.at[src_v], rows, sem).wait()

        def scale(g, _):
            w16 = 1.0 - d_v[pl.ds(g * 16, 16)]
            for j in range(16):
                w = w16[j]
                e = g * 16 + j
                for k in range(_D // 16):
                    rows[e, pl.ds(k * 16, 16)] = rows[e, pl.ds(k * 16, 16)] * w
            return 0
        lax.fori_loop(0, _C1 // 16, scale, 0)

        for g in range(_C1 // 16):
            dv = dst_v[pl.ds(g * 16, 16)]
            plsc.addupdate_scatter(deg_l, [dv], ones16)
            cv = cv + (dv < _HALF).astype(jnp.int32)

        pltpu.sync_copy(rows, acc_sh.at[dst_v], add=True)
        return cv
    cv = lax.fori_loop(0, _NC1, chunk, jnp.zeros((16,), jnp.int32))
    c0 = plsc.cumsum(cv)[15]

    def pchunk(i, bases):
        b0, b1 = bases
        base = eb + i * _C1
        pltpu.sync_copy(src_hbm.at[pl.ds(base, _C1)], src_v)
        pltpu.sync_copy(dst_hbm.at[pl.ds(base, _C1)], dst_v)
        pltpu.sync_copy(d_hbm.at[pl.ds(base, _C1)], d_v)
        for g in range(_C1 // 16):
            s16 = src_v[pl.ds(g * 16, 16)]
            t16 = dst_v[pl.ds(g * 16, 16)]
            w16 = d_v[pl.ds(g * 16, 16)]
            m = (t16 < _HALF).astype(jnp.int32)
            inv = 1 - m
            cs0 = plsc.cumsum(m)
            cs1 = plsc.cumsum(inv)
            pos = jnp.where(m == 1, b0 + cs0 - m, b1 + cs1 - inv)
            plsc.store_scatter(stg_s, [pos], s16)
            plsc.store_scatter(stg_t, [pos], t16)
            plsc.store_scatter(stg_w, [pos], w16)
            b0 = b0 + cs0[15]
            b1 = b1 + cs1[15]
        return (b0, b1)
    lax.fori_loop(0, _NC1, pchunk, (jnp.int32(0), c0))

    pltpu.sync_copy(stg_s, bsrc_out.at[pl.ds(wid * _EPT, _EPT)])
    pltpu.sync_copy(stg_t, bdst_out.at[pl.ds(wid * _EPT, _EPT)])
    pltpu.sync_copy(stg_w, bd_out.at[pl.ds(wid * _EPT, _EPT)])
    cvec[pl.ds(0, 16)] = jnp.zeros((16,), jnp.int32) + c0
    pltpu.sync_copy(cvec.at[pl.ds(0, 8)], cnt_out.at[pl.ds(wid * 8, 8)])

    @pl.when(wid == 31)
    def _():
        big16 = jnp.full((16,), _BIGDST, jnp.int32)
        zero16i = jnp.zeros((16,), jnp.int32)

        def dfill(i, _):
            stg_s[pl.ds(i * 16, 16)] = zero16i
            stg_t[pl.ds(i * 16, 16)] = big16
            stg_w[pl.ds(i * 16, 16)] = zero16
            return 0
        lax.fori_loop(0, _PAD // 16, dfill, 0)
        pltpu.sync_copy(stg_s.at[pl.ds(0, _PAD)], bsrc_out.at[pl.ds(_E, _PAD)])
        pltpu.sync_copy(stg_t.at[pl.ds(0, _PAD)], bdst_out.at[pl.ds(_E, _PAD)])
        pltpu.sync_copy(stg_w.at[pl.ds(0, _PAD)], bd_out.at[pl.ds(_E, _PAD)])

    pltpu.sync_copy(deg_l, deg_out.at[wid])
    plsc.subcore_barrier()

    pltpu.sync_copy(acc_sh.at[pl.ds(r0, _RPS), :],
                    acc_out.at[c, pl.ds(r0, _RPS), :])

    @pl.when(s == 15)
    def _():
        pltpu.sync_copy(acc_sh.at[pl.ds(16 * _RPS, _N - 16 * _RPS), :],
                        acc_out.at[c, pl.ds(16 * _RPS, _N - 16 * _RPS), :])


_RB = 1000


def _tc_mats_body(acc_ref, deg_ref, x_ref, w1_ref, b1_ref, wt_ref, wp_ref,
                  bpt_ref, ht_ref, hpb_ref):
    a = acc_ref[0] + acc_ref[1]
    degs = jnp.sum(deg_ref[...], axis=0)
    h0 = a + degs * x_ref[...]
    h = jnp.maximum(
        lax.dot_general(h0, w1_ref[...], (((1,), (1,)), ((), ())),
                        preferred_element_type=jnp.float32) + b1_ref[...],
        0.0)
    ht_ref[...] = lax.dot_general(h, wt_ref[...], (((1,), (1,)), ((), ())),
                                  preferred_element_type=jnp.float32)
    hpb_ref[...] = lax.dot_general(h, wp_ref[...], (((1,), (1,)), ((), ())),
                                   preferred_element_type=jnp.float32) + bpt_ref[...]


def _tc_mats(acc, deg, x, W1, b1, Wt, Wp, bpt):
    return pl.pallas_call(
        _tc_mats_body,
        grid=(_N // _RB,),
        in_specs=[
            pl.BlockSpec((2, _RB, _D), lambda i: (0, i, 0)),
            pl.BlockSpec((_NTILES, _RB, 1), lambda i: (0, i, 0)),
            pl.BlockSpec((_RB, _D), lambda i: (i, 0)),
            pl.BlockSpec((_D, _D), lambda i: (0, 0)),
            pl.BlockSpec((1, _D), lambda i: (0, 0)),
            pl.BlockSpec((_D, _D), lambda i: (0, 0)),
            pl.BlockSpec((_D, _D), lambda i: (0, 0)),
            pl.BlockSpec((1, _D), lambda i: (0, 0)),
        ],
        out_specs=[
            pl.BlockSpec((_RB, _D), lambda i: (i, 0)),
            pl.BlockSpec((_RB, _D), lambda i: (i, 0)),
        ],
        out_shape=[
            jax.ShapeDtypeStruct((_N, _D), jnp.float32),
            jax.ShapeDtypeStruct((_N, _D), jnp.float32),
        ],
    )(acc, deg, x, W1, b1, Wt, Wp, bpt)


@functools.partial(
    pl.kernel,
    out_type=jax.ShapeDtypeStruct((2, 16, _HALF * 16), jnp.float32),
    mesh=_mesh,
    compiler_params=_sc_params,
    scratch_types=(
        pltpu.VMEM((_C2,), jnp.int32),
        pltpu.VMEM((_C2,), jnp.int32),
        pltpu.VMEM((_C2,), jnp.float32),
        pltpu.VMEM((_NSUB, 128), jnp.int32),
        pltpu.VMEM((_C2, 16), jnp.float32),
        pltpu.VMEM((144,), jnp.int32),
        pltpu.VMEM((_ACCR * 16,), jnp.float32),
        pltpu.SemaphoreType.DMA,
    ),
)
def _stage2(htb_hbm, bsrc_hbm, bdst_hbm, bd_hbm, cnt_hbm, mx_out,
            src_v, dst_v, d_v, sidx, rows, cnts, acc, sem):
    c = lax.axis_index("c")
    s = lax.axis_index("s")
    fb = s % 8
    h = s // 8
    eq = c
    fbN = fb * _N
    negv = jnp.full((16,), _NEG, jnp.float32)

    def ini(i, _):
        acc[pl.ds(i * 16, 16)] = negv
        return 0
    lax.fori_loop(0, _ACCR, ini, 0)

    pltpu.sync_copy(cnt_hbm.at[pl.ds(eq * 128, 128)], cnts.at[pl.ds(0, 128)])
    h5 = h * _HALF

    def region(t, _):
        c16 = cnts[pl.ds(t * 8, 16)]
        c0 = c16[0]
        rbase = (eq * 16 + t) * _EPT
        c0a = (c0 // 8) * 8
        start = jnp.where(h == 0, rbase, rbase + c0a)
        ln = jnp.where(h == 0, c0, _EPT - c0a)
        nch = (ln + _C2 - 1) // _C2

        def chunk(i, _):
            base = start + i * _C2
            pltpu.sync_copy(bsrc_hbm.at[pl.ds(base, _C2)], src_v)
            pltpu.sync_copy(bdst_hbm.at[pl.ds(base, _C2)], dst_v)
            pltpu.sync_copy(bd_hbm.at[pl.ds(base, _C2)], d_v)

            def mkidx(k, _):
                sidx[k // 8, pl.ds((k % 8) * 16, 16)] = (
                    src_v[pl.ds(k * 16, 16)] + fbN)
                return 0
            lax.fori_loop(0, _C2 // 16, mkidx, 0)

            cps = [pltpu.async_copy(htb_hbm.at[sidx.at[k]],
                                    rows.at[pl.ds(k * 128, 128), :], sem)
                   for k in range(_NSUB)]
            for cp in cps:
                cp.wait()

            def grp(g, _):
                t16 = dst_v[pl.ds(g * 16, 16)]
                a16 = t16 - h5
                ok = (a16 >= 0) & (a16 < _HALF)
                addr16 = jnp.where(ok, a16, _TRASH) * 16
                dd16 = d_v[pl.ds(g * 16, 16)]
                for j in range(16):
                    aj = addr16[j]
                    dj = dd16[j]
                    e = g * 16 + j
                    rv = rows[e, pl.ds(0, 16)]
                    val = rv * dj
                    cur = acc[pl.ds(aj, 16)]
                    acc[pl.ds(aj, 16)] = jnp.maximum(cur, val)
                return 0
            lax.fori_loop(0, _C2 // 16, grp, 0)
            return 0
        lax.fori_loop(0, nch, chunk, 0)
        return 0
    lax.fori_loop(0, 16, region, 0)

    pltpu.sync_copy(acc.at[pl.ds(0, _HALF * 16)], mx_out.at[c, s])


def _tc_mean_body(m_ref, hpb_ref, out_ref):
    v = jnp.maximum(jnp.maximum(m_ref[0], m_ref[1]) + hpb_ref[...], 0.0)
    part = jnp.sum(v, axis=0, keepdims=True) * (1.0 / _N)

    @pl.when(pl.program_id(0) == 0)
    def _():
        out_ref[...] = part

    @pl.when(pl.program_id(0) != 0)
    def _():
        out_ref[...] = out_ref[...] + part


def _tc_mean(mt, hpb):
    return pl.pallas_call(
        _tc_mean_body,
        grid=(_N // _RB,),
        in_specs=[
            pl.BlockSpec((2, _RB, _D), lambda i: (0, i, 0)),
            pl.BlockSpec((_RB, _D), lambda i: (i, 0)),
        ],
        out_specs=pl.BlockSpec((1, _D), lambda i: (0, 0)),
        out_shape=jax.ShapeDtypeStruct((1, _D), jnp.float32),
    )(mt, hpb)


def kernel(x, edge_index, edge_d, W1, b1, Wt, bt, Wp, bp):
    src = edge_index[0]
    dst = edge_index[1]
    acc, deg, bsrc, bdst, bd, cnt = _stage1(x, src, dst, edge_d)
    ht, hpb = _tc_mats(acc, deg.reshape(_NTILES, _N, 1), x, W1,
                       b1.reshape(1, _D), Wt, Wp, (bt + bp).reshape(1, _D))
    htb = ht.reshape(_N, 8, 16).transpose(1, 0, 2).reshape(8 * _N, 16)
    m = _stage2(htb, bsrc, bdst, bd, cnt)
    m4 = (m.reshape(2, 2, 8, _HALF, 16)
           .transpose(0, 1, 3, 2, 4)
           .reshape(2, _N, _D))
    return _tc_mean(m4, hpb)


# speedup vs baseline: 3.0093x; 1.2106x over previous
import functools

import jax
import jax.numpy as jnp
from jax import lax
from jax.experimental import pallas as pl
from jax.experimental.pallas import tpu as pltpu
from jax.experimental.pallas import tpu_sc as plsc

_N = 10000
_E = 320000
_D = 128

_NTILES = 32
_EPT = _E // _NTILES
_C1 = 80
_NC1 = _EPT // _C1
_RPS = 624

_HALF = _N // 2
_PAD = 512
_C2 = 512
_NSUB = _C2 // 128
_TRASH = _HALF
_ACCR = 5008
_NEG = -3.0e38
_BIGDST = 1 << 29

_mesh = plsc.VectorSubcoreMesh(core_axis_name="c", subcore_axis_name="s")
_sc_params = pltpu.CompilerParams(needs_layout_passes=False,
                                  use_tc_tiling_on_sc=False)


@functools.partial(
    pl.kernel,
    out_type=(jax.ShapeDtypeStruct((2, _N, _D), jnp.float32),
              jax.ShapeDtypeStruct((_NTILES, _N), jnp.float32),
              jax.ShapeDtypeStruct((_E + _PAD,), jnp.int32),
              jax.ShapeDtypeStruct((_E + _PAD,), jnp.int32),
              jax.ShapeDtypeStruct((_E + _PAD,), jnp.float32),
              jax.ShapeDtypeStruct((_NTILES * 8,), jnp.int32)),
    mesh=_mesh,
    compiler_params=_sc_params,
    scratch_types=(
        pltpu.VMEM((_C1,), jnp.int32),
        pltpu.VMEM((_C1,), jnp.int32),
        pltpu.VMEM((_C1,), jnp.float32),
        pltpu.VMEM((_C1, _D), jnp.float32),
        pltpu.VMEM((_N,), jnp.float32),
        pltpu.VMEM((_EPT,), jnp.int32),
        pltpu.VMEM((_EPT,), jnp.int32),
        pltpu.VMEM((_EPT,), jnp.float32),
        pltpu.VMEM((16,), jnp.int32),
        pltpu.VMEM_SHARED((_N, _D), jnp.float32),
        pltpu.SemaphoreType.DMA,
    ),
)
def _stage1(x_hbm, src_hbm, dst_hbm, d_hbm,
            acc_out, deg_out, bsrc_out, bdst_out, bd_out, cnt_out,
            src_v, dst_v, d_v, rows, deg_l, stg_s, stg_t, stg_w, cvec,
            acc_sh, sem):
    c = lax.axis_index("c")
    s = lax.axis_index("s")
    wid = c * 16 + s
    zero16 = jnp.zeros((16,), jnp.float32)
    ones16 = jnp.ones((16,), jnp.float32)

    def zdeg(i, _):
        deg_l[pl.ds(i * 16, 16)] = zero16
        return 0
    lax.fori_loop(0, _N // 16, zdeg, 0)

    def zrows(j, _):
        for k in range(_D // 16):
            rows[j, pl.ds(k * 16, 16)] = zero16
        return 0
    lax.fori_loop(0, _C1, zrows, 0)

    r0 = s * _RPS
    for t in range(_RPS // _C1):
        pltpu.sync_copy(rows, acc_sh.at[pl.ds(r0 + t * _C1, _C1), :])
    rem = _RPS % _C1
    pltpu.sync_copy(rows.at[pl.ds(0, rem), :],
                    acc_sh.at[pl.ds(r0 + _RPS - rem, rem), :])

    @pl.when(s == 15)
    def _():
        pltpu.sync_copy(rows.at[pl.ds(0, _N - 16 * _RPS), :],
                        acc_sh.at[pl.ds(16 * _RPS, _N - 16 * _RPS), :])

    plsc.subcore_barrier()

    eb = wid * _EPT

    def chunk(i, bases):
        b0, b1 = bases
        base = eb + i * _C1
        pltpu.sync_copy(src_hbm.at[pl.ds(base, _C1)], src_v)
        pltpu.sync_copy(dst_hbm.at[pl.ds(base, _C1)], dst_v)
        pltpu.sync_copy(d_hbm.at[pl.ds(base, _C1)], d_v)
        pltpu.async_copy(x_hbm.at[src_v], rows, sem).wait()

        def scale(g, _):
            w16 = 1.0 - d_v[pl.ds(g * 16, 16)]
            for j in range(16):
                w = w16[j]
                e = g * 16 + j
                for k in range(_D // 16):
                    rows[e, pl.ds(k * 16, 16)] = rows[e, pl.ds(k * 16, 16)] * w
            return 0
        lax.fori_loop(0, _C1 // 16, scale, 0)

        for g in range(_C1 // 16):
            s16 = src_v[pl.ds(g * 16, 16)]
            t16 = dst_v[pl.ds(g * 16, 16)]
            w16 = d_v[pl.ds(g * 16, 16)]
            plsc.addupdate_scatter(deg_l, [t16], ones16)
            mi = (t16 < _HALF).astype(jnp.int32)
            inv = 1 - mi
            cs0 = plsc.cumsum(mi)
            cs1 = plsc.cumsum(inv)
            pos = jnp.where(mi == 1, b0 + cs0 - mi, _EPT - b1 - cs1)
            plsc.store_scatter(stg_s, [pos], s16)
            plsc.store_scatter(stg_t, [pos], t16)
            plsc.store_scatter(stg_w, [pos], w16)
            b0 = b0 + cs0[15]
            b1 = b1 + cs1[15]

        pltpu.sync_copy(rows, acc_sh.at[dst_v], add=True)
        return (b0, b1)
    c0, _unused_b1 = lax.fori_loop(0, _NC1, chunk,
                                   (jnp.int32(0), jnp.int32(0)))

    pltpu.sync_copy(stg_s, bsrc_out.at[pl.ds(wid * _EPT, _EPT)])
    pltpu.sync_copy(stg_t, bdst_out.at[pl.ds(wid * _EPT, _EPT)])
    pltpu.sync_copy(stg_w, bd_out.at[pl.ds(wid * _EPT, _EPT)])
    cvec[pl.ds(0, 16)] = jnp.zeros((16,), jnp.int32) + c0
    pltpu.sync_copy(cvec.at[pl.ds(0, 8)], cnt_out.at[pl.ds(wid * 8, 8)])

    @pl.when(wid == 31)
    def _():
        big16 = jnp.full((16,), _BIGDST, jnp.int32)
        zero16i = jnp.zeros((16,), jnp.int32)

        def dfill(i, _):
            stg_s[pl.ds(i * 16, 16)] = zero16i
            stg_t[pl.ds(i * 16, 16)] = big16
            stg_w[pl.ds(i * 16, 16)] = zero16
            return 0
        lax.fori_loop(0, _PAD // 16, dfill, 0)
        pltpu.sync_copy(stg_s.at[pl.ds(0, _PAD)], bsrc_out.at[pl.ds(_E, _PAD)])
        pltpu.sync_copy(stg_t.at[pl.ds(0, _PAD)], bdst_out.at[pl.ds(_E, _PAD)])
        pltpu.sync_copy(stg_w.at[pl.ds(0, _PAD)], bd_out.at[pl.ds(_E, _PAD)])

    pltpu.sync_copy(deg_l, deg_out.at[wid])
    plsc.subcore_barrier()

    pltpu.sync_copy(acc_sh.at[pl.ds(r0, _RPS), :],
                    acc_out.at[c, pl.ds(r0, _RPS), :])

    @pl.when(s == 15)
    def _():
        pltpu.sync_copy(acc_sh.at[pl.ds(16 * _RPS, _N - 16 * _RPS), :],
                        acc_out.at[c, pl.ds(16 * _RPS, _N - 16 * _RPS), :])


_RB = 1000


def _tc_mats_body(acc_ref, deg_ref, x_ref, w1_ref, b1_ref, wt_ref, wp_ref,
                  bpt_ref, ht_ref, hpb_ref):
    a = acc_ref[0] + acc_ref[1]
    degs = jnp.sum(deg_ref[...], axis=0)
    h0 = a + degs * x_ref[...]
    h = jnp.maximum(
        lax.dot_general(h0, w1_ref[...], (((1,), (1,)), ((), ())),
                        preferred_element_type=jnp.float32) + b1_ref[...],
        0.0)
    ht_ref[...] = lax.dot_general(h, wt_ref[...], (((1,), (1,)), ((), ())),
                                  preferred_element_type=jnp.float32)
    hpb_ref[...] = lax.dot_general(h, wp_ref[...], (((1,), (1,)), ((), ())),
                                   preferred_element_type=jnp.float32) + bpt_ref[...]


def _tc_mats(acc, deg, x, W1, b1, Wt, Wp, bpt):
    return pl.pallas_call(
        _tc_mats_body,
        grid=(_N // _RB,),
        in_specs=[
            pl.BlockSpec((2, _RB, _D), lambda i: (0, i, 0)),
            pl.BlockSpec((_NTILES, _RB, 1), lambda i: (0, i, 0)),
            pl.BlockSpec((_RB, _D), lambda i: (i, 0)),
            pl.BlockSpec((_D, _D), lambda i: (0, 0)),
            pl.BlockSpec((1, _D), lambda i: (0, 0)),
            pl.BlockSpec((_D, _D), lambda i: (0, 0)),
            pl.BlockSpec((_D, _D), lambda i: (0, 0)),
            pl.BlockSpec((1, _D), lambda i: (0, 0)),
        ],
        out_specs=[
            pl.BlockSpec((_RB, _D), lambda i: (i, 0)),
            pl.BlockSpec((_RB, _D), lambda i: (i, 0)),
        ],
        out_shape=[
            jax.ShapeDtypeStruct((_N, _D), jnp.float32),
            jax.ShapeDtypeStruct((_N, _D), jnp.float32),
        ],
    )(acc, deg, x, W1, b1, Wt, Wp, bpt)


@functools.partial(
    pl.kernel,
    out_type=jax.ShapeDtypeStruct((2, 16, _HALF * 16), jnp.float32),
    mesh=_mesh,
    compiler_params=_sc_params,
    scratch_types=(
        pltpu.VMEM((_C2,), jnp.int32),
        pltpu.VMEM((2 * _C2,), jnp.int32),
        pltpu.VMEM((2 * _C2,), jnp.float32),
        pltpu.VMEM((2 * _NSUB, 128), jnp.int32),
        pltpu.VMEM((2 * _C2, 16), jnp.float32),
        pltpu.VMEM((144,), jnp.int32),
        pltpu.VMEM((_ACCR * 16,), jnp.float32),
        pltpu.SemaphoreType.DMA,
    ),
)
def _stage2(htb_hbm, bsrc_hbm, bdst_hbm, bd_hbm, cnt_hbm, mx_out,
            src_v, dst_v, d_v, sidx, rows, cnts, acc, sem):
    c = lax.axis_index("c")
    s = lax.axis_index("s")
    fb = s % 8
    h = s // 8
    eq = c
    fbN = fb * _N
    negv = jnp.full((16,), _NEG, jnp.float32)

    def ini(i, _):
        acc[pl.ds(i * 16, 16)] = negv
        return 0
    lax.fori_loop(0, _ACCR, ini, 0)

    pltpu.sync_copy(cnt_hbm.at[pl.ds(eq * 128, 128)], cnts.at[pl.ds(0, 128)])
    h5 = h * _HALF

    def fetch(base, p):
        pltpu.sync_copy(bsrc_hbm.at[pl.ds(base, _C2)], src_v)
        pltpu.sync_copy(bdst_hbm.at[pl.ds(base, _C2)],
                        dst_v.at[pl.ds(p * _C2, _C2)])
        pltpu.sync_copy(bd_hbm.at[pl.ds(base, _C2)],
                        d_v.at[pl.ds(p * _C2, _C2)])

        def mkidx(k, _):
            sidx[p * _NSUB + k // 8, pl.ds((k % 8) * 16, 16)] = (
                src_v[pl.ds(k * 16, 16)] + fbN)
            return 0
        lax.fori_loop(0, _C2 // 16, mkidx, 0)
        for k in range(_NSUB):
            pltpu.async_copy(htb_hbm.at[sidx.at[p * _NSUB + k]],
                             rows.at[pl.ds((p * _NSUB + k) * 128, 128), :],
                             sem)

    def drain(p):
        for k in range(_NSUB):
            pltpu.make_async_copy(
                htb_hbm.at[sidx.at[p * _NSUB + k]],
                rows.at[pl.ds((p * _NSUB + k) * 128, 128), :],
                sem).wait()

    def region(t, _):
        c16 = cnts[pl.ds(t * 8, 16)]
        c0 = c16[0]
        rbase = (eq * 16 + t) * _EPT
        c0a = (c0 // 8) * 8
        start = jnp.where(h == 0, rbase, rbase + c0a)
        ln = jnp.where(h == 0, c0, _EPT - c0a)
        nch = (ln + _C2 - 1) // _C2

        @pl.when(nch > 0)
        def _():
            fetch(start, 0)

            def chunk(i, _):
                p = lax.rem(i, 2)

                @pl.when(i + 1 < nch)
                def _():
                    fetch(start + (i + 1) * _C2, 1 - p)

                drain(p)

                def grp(g, _):
                    t16 = dst_v[pl.ds(p * _C2 + g * 16, 16)]
                    a16 = t16 - h5
                    ok = (a16 >= 0) & (a16 < _HALF)
                    addr16 = jnp.where(ok, a16, _TRASH) * 16
                    dd16 = d_v[pl.ds(p * _C2 + g * 16, 16)]
                    for j in range(16):
                        aj = addr16[j]
                        dj = dd16[j]
                        e = p * _C2 + g * 16 + j
                        rv = rows[e, pl.ds(0, 16)]
                        val = rv * dj
                        cur = acc[pl.ds(aj, 16)]
                        acc[pl.ds(aj, 16)] = jnp.maximum(cur, val)
                    return 0
                lax.fori_loop(0, _C2 // 16, grp, 0)
                return 0
            lax.fori_loop(0, nch, chunk, 0)
        return 0
    lax.fori_loop(0, 16, region, 0)

    pltpu.sync_copy(acc.at[pl.ds(0, _HALF * 16)], mx_out.at[c, s])


def _tc_mean_body(m_ref, hpb_ref, out_ref):
    v = jnp.maximum(jnp.maximum(m_ref[0], m_ref[1]) + hpb_ref[...], 0.0)
    part = jnp.sum(v, axis=0, keepdims=True) * (1.0 / _N)

    @pl.when(pl.program_id(0) == 0)
    def _():
        out_ref[...] = part

    @pl.when(pl.program_id(0) != 0)
    def _():
        out_ref[...] = out_ref[...] + part


def _tc_mean(mt, hpb):
    return pl.pallas_call(
        _tc_mean_body,
        grid=(_N // _RB,),
        in_specs=[
            pl.BlockSpec((2, _RB, _D), lambda i: (0, i, 0)),
            pl.BlockSpec((_RB, _D), lambda i: (i, 0)),
        ],
        out_specs=pl.BlockSpec((1, _D), lambda i: (0, 0)),
        out_shape=jax.ShapeDtypeStruct((1, _D), jnp.float32),
    )(mt, hpb)


def kernel(x, edge_index, edge_d, W1, b1, Wt, bt, Wp, bp):
    src = edge_index[0]
    dst = edge_index[1]
    acc, deg, bsrc, bdst, bd, cnt = _stage1(x, src, dst, edge_d)
    ht, hpb = _tc_mats(acc, deg.reshape(_NTILES, _N, 1), x, W1,
                       b1.reshape(1, _D), Wt, Wp, (bt + bp).reshape(1, _D))
    htb = ht.reshape(_N, 8, 16).transpose(1, 0, 2).reshape(8 * _N, 16)
    m = _stage2(htb, bsrc, bdst, bd, cnt)
    m4 = (m.reshape(2, 2, 8, _HALF, 16)
           .transpose(0, 1, 3, 2, 4)
           .reshape(2, _N, _D))
    return _tc_mean(m4, hpb)


# speedup vs baseline: 3.4680x; 1.1524x over previous
import functools

import jax
import jax.numpy as jnp
from jax import lax
from jax.experimental import pallas as pl
from jax.experimental.pallas import tpu as pltpu
from jax.experimental.pallas import tpu_sc as plsc

_N = 10000
_E = 320000
_D = 128

_NTILES = 32
_EPT = _E // _NTILES
_C1 = 80
_NC1 = _EPT // _C1
_RPS = 624

_HALF = _N // 2
_PAD = 512
_C2 = 512
_NSUB = _C2 // 128
_TRASH = _HALF
_ACCR = 5008
_NEG = -3.0e38
_BIGDST = 1 << 29

_mesh = plsc.VectorSubcoreMesh(core_axis_name="c", subcore_axis_name="s")
_sc_params = pltpu.CompilerParams(needs_layout_passes=False,
                                  use_tc_tiling_on_sc=False)


@functools.partial(
    pl.kernel,
    out_type=(jax.ShapeDtypeStruct((2, _N, _D), jnp.float32),
              jax.ShapeDtypeStruct((_NTILES, _N), jnp.float32),
              jax.ShapeDtypeStruct((_E + _PAD,), jnp.int32),
              jax.ShapeDtypeStruct((_E + _PAD,), jnp.int32),
              jax.ShapeDtypeStruct((_E + _PAD,), jnp.float32),
              jax.ShapeDtypeStruct((_NTILES * 8,), jnp.int32)),
    mesh=_mesh,
    compiler_params=_sc_params,
    scratch_types=(
        pltpu.VMEM((_C1,), jnp.int32),
        pltpu.VMEM((_C1,), jnp.int32),
        pltpu.VMEM((_C1,), jnp.float32),
        pltpu.VMEM((_C1, _D), jnp.float32),
        pltpu.VMEM((_N,), jnp.float32),
        pltpu.VMEM((_EPT,), jnp.int32),
        pltpu.VMEM((_EPT,), jnp.int32),
        pltpu.VMEM((_EPT,), jnp.float32),
        pltpu.VMEM((16,), jnp.int32),
        pltpu.VMEM_SHARED((_N, _D), jnp.float32),
        pltpu.SemaphoreType.DMA,
    ),
)
def _stage1(x_hbm, src_hbm, dst_hbm, d_hbm,
            acc_out, deg_out, bsrc_out, bdst_out, bd_out, cnt_out,
            src_v, dst_v, d_v, rows, deg_l, stg_s, stg_t, stg_w, cvec,
            acc_sh, sem):
    c = lax.axis_index("c")
    s = lax.axis_index("s")
    wid = c * 16 + s
    zero16 = jnp.zeros((16,), jnp.float32)
    ones16 = jnp.ones((16,), jnp.float32)

    def zdeg(i, _):
        deg_l[pl.ds(i * 16, 16)] = zero16
        return 0
    lax.fori_loop(0, _N // 16, zdeg, 0)

    def zrows(j, _):
        for k in range(_D // 16):
            rows[j, pl.ds(k * 16, 16)] = zero16
        return 0
    lax.fori_loop(0, _C1, zrows, 0)

    r0 = s * _RPS
    for t in range(_RPS // _C1):
        pltpu.sync_copy(rows, acc_sh.at[pl.ds(r0 + t * _C1, _C1), :])
    rem = _RPS % _C1
    pltpu.sync_copy(rows.at[pl.ds(0, rem), :],
                    acc_sh.at[pl.ds(r0 + _RPS - rem, rem), :])

    @pl.when(s == 15)
    def _():
        pltpu.sync_copy(rows.at[pl.ds(0, _N - 16 * _RPS), :],
                        acc_sh.at[pl.ds(16 * _RPS, _N - 16 * _RPS), :])

    plsc.subcore_barrier()

    eb = wid * _EPT

    def chunk(i, bases):
        b0, b1 = bases
        base = eb + i * _C1
        pltpu.sync_copy(src_hbm.at[pl.ds(base, _C1)], src_v)
        pltpu.sync_copy(dst_hbm.at[pl.ds(base, _C1)], dst_v)
        pltpu.sync_copy(d_hbm.at[pl.ds(base, _C1)], d_v)
        pltpu.async_copy(x_hbm.at[src_v], rows, sem).wait()

        def scale(g, _):
            w16 = 1.0 - d_v[pl.ds(g * 16, 16)]
            for j in range(16):
                w = w16[j]
                e = g * 16 + j
                for k in range(_D // 16):
                    rows[e, pl.ds(k * 16, 16)] = rows[e, pl.ds(k * 16, 16)] * w
            return 0
        lax.fori_loop(0, _C1 // 16, scale, 0)

        for g in range(_C1 // 16):
            s16 = src_v[pl.ds(g * 16, 16)]
            t16 = dst_v[pl.ds(g * 16, 16)]
            w16 = d_v[pl.ds(g * 16, 16)]
            plsc.addupdate_scatter(deg_l, [t16], ones16)
            mi = (t16 < _HALF).astype(jnp.int32)
            inv = 1 - mi
            cs0 = plsc.cumsum(mi)
            cs1 = plsc.cumsum(inv)
            pos = jnp.where(mi == 1, b0 + cs0 - mi, _EPT - b1 - cs1)
            plsc.store_scatter(stg_s, [pos], s16)
            plsc.store_scatter(stg_t, [pos], t16)
            plsc.store_scatter(stg_w, [pos], w16)
            b0 = b0 + cs0[15]
            b1 = b1 + cs1[15]

        pltpu.sync_copy(rows, acc_sh.at[dst_v], add=True)
        return (b0, b1)
    c0, _unused_b1 = lax.fori_loop(0, _NC1, chunk,
                                   (jnp.int32(0), jnp.int32(0)))

    pltpu.sync_copy(stg_s, bsrc_out.at[pl.ds(wid * _EPT, _EPT)])
    pltpu.sync_copy(stg_t, bdst_out.at[pl.ds(wid * _EPT, _EPT)])
    pltpu.sync_copy(stg_w, bd_out.at[pl.ds(wid * _EPT, _EPT)])
    cvec[pl.ds(0, 16)] = jnp.zeros((16,), jnp.int32) + c0
    pltpu.sync_copy(cvec.at[pl.ds(0, 8)], cnt_out.at[pl.ds(wid * 8, 8)])

    @pl.when(wid == 31)
    def _():
        big16 = jnp.full((16,), _BIGDST, jnp.int32)
        zero16i = jnp.zeros((16,), jnp.int32)

        def dfill(i, _):
            stg_s[pl.ds(i * 16, 16)] = zero16i
            stg_t[pl.ds(i * 16, 16)] = big16
            stg_w[pl.ds(i * 16, 16)] = zero16
            return 0
        lax.fori_loop(0, _PAD // 16, dfill, 0)
        pltpu.sync_copy(stg_s.at[pl.ds(0, _PAD)], bsrc_out.at[pl.ds(_E, _PAD)])
        pltpu.sync_copy(stg_t.at[pl.ds(0, _PAD)], bdst_out.at[pl.ds(_E, _PAD)])
        pltpu.sync_copy(stg_w.at[pl.ds(0, _PAD)], bd_out.at[pl.ds(_E, _PAD)])

    pltpu.sync_copy(deg_l, deg_out.at[wid])
    plsc.subcore_barrier()

    pltpu.sync_copy(acc_sh.at[pl.ds(r0, _RPS), :],
                    acc_out.at[c, pl.ds(r0, _RPS), :])

    @pl.when(s == 15)
    def _():
        pltpu.sync_copy(acc_sh.at[pl.ds(16 * _RPS, _N - 16 * _RPS), :],
                        acc_out.at[c, pl.ds(16 * _RPS, _N - 16 * _RPS), :])


_RB = 1000


def _tc_mats_body(acc_ref, deg_ref, x_ref, w1_ref, b1_ref, wt_ref, wp_ref,
                  bpt_ref, ht_ref, hpb_ref):
    a = acc_ref[0] + acc_ref[1]
    degs = jnp.sum(deg_ref[...], axis=0)
    h0 = a + degs * x_ref[...]
    h = jnp.maximum(
        lax.dot_general(h0, w1_ref[...], (((1,), (1,)), ((), ())),
                        preferred_element_type=jnp.float32) + b1_ref[...],
        0.0)
    ht_ref[...] = lax.dot_general(h, wt_ref[...], (((1,), (1,)), ((), ())),
                                  preferred_element_type=jnp.float32)
    hpb_ref[...] = lax.dot_general(h, wp_ref[...], (((1,), (1,)), ((), ())),
                                   preferred_element_type=jnp.float32) + bpt_ref[...]


def _tc_mats(acc, deg, x, W1, b1, Wt, Wp, bpt):
    return pl.pallas_call(
        _tc_mats_body,
        grid=(_N // _RB,),
        in_specs=[
            pl.BlockSpec((2, _RB, _D), lambda i: (0, i, 0)),
            pl.BlockSpec((_NTILES, _RB, 1), lambda i: (0, i, 0)),
            pl.BlockSpec((_RB, _D), lambda i: (i, 0)),
            pl.BlockSpec((_D, _D), lambda i: (0, 0)),
            pl.BlockSpec((1, _D), lambda i: (0, 0)),
            pl.BlockSpec((_D, _D), lambda i: (0, 0)),
            pl.BlockSpec((_D, _D), lambda i: (0, 0)),
            pl.BlockSpec((1, _D), lambda i: (0, 0)),
        ],
        out_specs=[
            pl.BlockSpec((_RB, _D), lambda i: (i, 0)),
            pl.BlockSpec((_RB, _D), lambda i: (i, 0)),
        ],
        out_shape=[
            jax.ShapeDtypeStruct((_N, _D), jnp.float32),
            jax.ShapeDtypeStruct((_N, _D), jnp.float32),
        ],
    )(acc, deg, x, W1, b1, Wt, Wp, bpt)


@functools.partial(
    pl.kernel,
    out_type=jax.ShapeDtypeStruct((2, 16, _HALF * 16), jnp.float32),
    mesh=_mesh,
    compiler_params=_sc_params,
    scratch_types=(
        pltpu.VMEM((_C2,), jnp.int32),
        pltpu.VMEM((3 * _C2,), jnp.int32),
        pltpu.VMEM((3 * _C2,), jnp.float32),
        pltpu.VMEM((2 * _NSUB, 128), jnp.int32),
        pltpu.VMEM((2 * _C2, 16), jnp.float32),
        pltpu.VMEM((144,), jnp.int32),
        pltpu.VMEM((_ACCR * 16,), jnp.float32),
        pltpu.SemaphoreType.DMA,
        pltpu.SemaphoreType.DMA,
    ),
)
def _stage2(htb_hbm, bsrc_hbm, bdst_hbm, bd_hbm, cnt_hbm, mx_out,
            src_v, dst_v, d_v, sidx, rows, cnts, acc, sem, sem2):
    c = lax.axis_index("c")
    s = lax.axis_index("s")
    fb = s % 8
    h = s // 8
    eq = c
    fbN = fb * _N
    negv = jnp.full((16,), _NEG, jnp.float32)

    def ini(i, _):
        acc[pl.ds(i * 16, 16)] = negv
        return 0
    lax.fori_loop(0, _ACCR, ini, 0)

    pltpu.sync_copy(cnt_hbm.at[pl.ds(eq * 128, 128)], cnts.at[pl.ds(0, 128)])
    h5 = h * _HALF

    def fire_linear(base, q):
        pltpu.async_copy(bsrc_hbm.at[pl.ds(base, _C2)], src_v, sem2)
        pltpu.async_copy(bdst_hbm.at[pl.ds(base, _C2)],
                         dst_v.at[pl.ds(q * _C2, _C2)], sem2)
        pltpu.async_copy(bd_hbm.at[pl.ds(base, _C2)],
                         d_v.at[pl.ds(q * _C2, _C2)], sem2)

    def drain_linear(base, q):
        pltpu.make_async_copy(bsrc_hbm.at[pl.ds(base, _C2)], src_v,
                              sem2).wait()
        pltpu.make_async_copy(bdst_hbm.at[pl.ds(base, _C2)],
                              dst_v.at[pl.ds(q * _C2, _C2)], sem2).wait()
        pltpu.make_async_copy(bd_hbm.at[pl.ds(base, _C2)],
                              d_v.at[pl.ds(q * _C2, _C2)], sem2).wait()

    def fire_gather(p):
        def mkidx(k, _):
            sidx[p * _NSUB + k // 8, pl.ds((k % 8) * 16, 16)] = (
                src_v[pl.ds(k * 16, 16)] + fbN)
            return 0
        lax.fori_loop(0, _C2 // 16, mkidx, 0)
        for k in range(_NSUB):
            pltpu.async_copy(htb_hbm.at[sidx.at[p * _NSUB + k]],
                             rows.at[pl.ds((p * _NSUB + k) * 128, 128), :],
                             sem)

    def drain_gather(p):
        for k in range(_NSUB):
            pltpu.make_async_copy(
                htb_hbm.at[sidx.at[p * _NSUB + k]],
                rows.at[pl.ds((p * _NSUB + k) * 128, 128), :],
                sem).wait()

    def region(t, _):
        c16 = cnts[pl.ds(t * 8, 16)]
        c0 = c16[0]
        rbase = (eq * 16 + t) * _EPT
        c0a = (c0 // 8) * 8
        start = jnp.where(h == 0, rbase, rbase + c0a)
        ln = jnp.where(h == 0, c0, _EPT - c0a)
        nch = (ln + _C2 - 1) // _C2

        @pl.when(nch > 0)
        def _():
            fire_linear(start, 0)
            drain_linear(start, 0)
            fire_gather(0)

            @pl.when(nch > 1)
            def _():
                fire_linear(start + _C2, 1)

            def chunk(i, _):
                p = lax.rem(i, 2)

                @pl.when(i + 1 < nch)
                def _():
                    q1 = lax.rem(i + 1, 3)
                    drain_linear(start + (i + 1) * _C2, q1)
                    fire_gather(1 - p)

                @pl.when(i + 2 < nch)
                def _():
                    fire_linear(start + (i + 2) * _C2, lax.rem(i + 2, 3))

                drain_gather(p)
                q = lax.rem(i, 3)

                def grp(g, _):
                    t16 = dst_v[pl.ds(q * _C2 + g * 16, 16)]
                    a16 = t16 - h5
                    ok = (a16 >= 0) & (a16 < _HALF)
                    addr16 = jnp.where(ok, a16, _TRASH) * 16
                    dd16 = d_v[pl.ds(q * _C2 + g * 16, 16)]
                    for j in range(16):
                        aj = addr16[j]
                        dj = dd16[j]
                        e = p * _C2 + g * 16 + j
                        rv = rows[e, pl.ds(0, 16)]
                        val = rv * dj
                        cur = acc[pl.ds(aj, 16)]
                        acc[pl.ds(aj, 16)] = jnp.maximum(cur, val)
                    return 0
                lax.fori_loop(0, _C2 // 16, grp, 0)
                return 0
            lax.fori_loop(0, nch, chunk, 0)
        return 0
    lax.fori_loop(0, 16, region, 0)

    pltpu.sync_copy(acc.at[pl.ds(0, _HALF * 16)], mx_out.at[c, s])


def _tc_mean_body(m_ref, hpb_ref, out_ref):
    v = jnp.maximum(jnp.maximum(m_ref[0], m_ref[1]) + hpb_ref[...], 0.0)
    part = jnp.sum(v, axis=0, keepdims=True) * (1.0 / _N)

    @pl.when(pl.program_id(0) == 0)
    def _():
        out_ref[...] = part

    @pl.when(pl.program_id(0) != 0)
    def _():
        out_ref[...] = out_ref[...] + part


def _tc_mean(mt, hpb):
    return pl.pallas_call(
        _tc_mean_body,
        grid=(_N // _RB,),
        in_specs=[
            pl.BlockSpec((2, _RB, _D), lambda i: (0, i, 0)),
            pl.BlockSpec((_RB, _D), lambda i: (i, 0)),
        ],
        out_specs=pl.BlockSpec((1, _D), lambda i: (0, 0)),
        out_shape=jax.ShapeDtypeStruct((1, _D), jnp.float32),
    )(mt, hpb)


def kernel(x, edge_index, edge_d, W1, b1, Wt, bt, Wp, bp):
    src = edge_index[0]
    dst = edge_index[1]
    acc, deg, bsrc, bdst, bd, cnt = _stage1(x, src, dst, edge_d)
    ht, hpb = _tc_mats(acc, deg.reshape(_NTILES, _N, 1), x, W1,
                       b1.reshape(1, _D), Wt, Wp, (bt + bp).reshape(1, _D))
    htb = ht.reshape(_N, 8, 16).transpose(1, 0, 2).reshape(8 * _N, 16)
    m = _stage2(htb, bsrc, bdst, bd, cnt)
    m4 = (m.reshape(2, 2, 8, _HALF, 16)
           .transpose(0, 1, 3, 2, 4)
           .reshape(2, _N, _D))
    return _tc_mean(m4, hpb)
